# baseline (speedup 1.0000x reference)
import functools

import jax
import jax.numpy as jnp
from jax import lax
from jax.experimental import pallas as pl
from jax.experimental.pallas import tpu as pltpu
from jax.experimental.pallas import tpu_sc as plsc

N = 10000
D = 128
E = 320000
DH = D // 2

NC = 2
NS = 16
NW = NC * NS
ECHUNK = 1000
NCH_DEG = E // NW // ECHUNK
NCH = E // NS // ECHUNK
NHALF = 2
NCH_H = NCH // NHALF
NBUF = 2
KFIRE = 5
STRIPE = 624
TAIL = N - NS * STRIPE

_MESH = plsc.VectorSubcoreMesh(
    core_axis_name="c", subcore_axis_name="s", num_cores=NC, num_subcores=NS
)


@functools.partial(
    pl.kernel,
    out_type=jax.ShapeDtypeStruct((NC, N, 16), jnp.float32),
    mesh=_MESH,
    compiler_params=pltpu.CompilerParams(use_tc_tiling_on_sc=False),
    scratch_types=[
        pltpu.VMEM((STRIPE, 16), jnp.float32),
        pltpu.VMEM((ECHUNK, 16), jnp.float32),
        pltpu.VMEM((NCH_DEG, 1, ECHUNK), jnp.int32),
        pltpu.VMEM_SHARED((N, 16), jnp.float32),
        pltpu.SemaphoreType.DMA,
    ],
)
def _deg_kernel(edge_hbm, out_hbm, z_v, o_v, dst_v, acc, sem):
    cid = lax.axis_index("c")
    sid = lax.axis_index("s")
    row0 = sid * STRIPE

    def zbody(i, _):
        z_v[i] = jnp.zeros((16,), jnp.float32)
        return 0
    lax.fori_loop(0, STRIPE, zbody, 0)

    def obody(i, _):
        o_v[i] = jnp.ones((16,), jnp.float32)
        return 0
    lax.fori_loop(0, ECHUNK, obody, 0)

    pltpu.sync_copy(z_v, acc.at[pl.ds(row0, STRIPE)])

    @pl.when(sid == 0)
    def _():
        pltpu.sync_copy(z_v.at[pl.ds(0, TAIL)], acc.at[pl.ds(NS * STRIPE, TAIL)])

    pltpu.sync_copy(edge_hbm.at[1, sid, pl.ds(cid * NCH_DEG, NCH_DEG)], dst_v)
    plsc.subcore_barrier()

    for g in range(NCH_DEG // KFIRE):
        for k in range(KFIRE):
            pltpu.async_copy(o_v, acc.at[dst_v.at[g * KFIRE + k, 0]],
                             sem, add=True)
        for k in range(KFIRE):
            pltpu.make_async_copy(o_v, acc.at[dst_v.at[g * KFIRE + k, 0]],
                                  sem).wait()

    plsc.subcore_barrier()
    pltpu.sync_copy(
        acc.at[pl.ds(row0, STRIPE)],
        out_hbm.at[cid, pl.ds(row0, STRIPE)],
    )

    @pl.when(sid == 0)
    def _():
        pltpu.sync_copy(
            acc.at[pl.ds(NS * STRIPE, TAIL)],
            out_hbm.at[cid, pl.ds(NS * STRIPE, TAIL)],
        )


@functools.partial(
    pl.kernel,
    out_type=jax.ShapeDtypeStruct((NC, NHALF, N, DH), jnp.bfloat16),
    mesh=_MESH,
    compiler_params=pltpu.CompilerParams(use_tc_tiling_on_sc=False),
    scratch_types=[
        pltpu.VMEM((NCH_H, 1, ECHUNK), jnp.int32),
        pltpu.VMEM((NCH_H, 1, ECHUNK), jnp.int32),
        [pltpu.VMEM((ECHUNK, DH), jnp.bfloat16) for _ in range(NBUF)],
        [pltpu.SemaphoreType.DMA for _ in range(NBUF)],
        [pltpu.VMEM_SHARED((N, DH), jnp.bfloat16) for _ in range(NHALF)],
    ],
)
def _edge_kernel(y_hbm, edge_hbm, out_hbm,
                 src_v, dst_v, rows, gsems, accs):
    cid = lax.axis_index("c")
    sid = lax.axis_index("s")
    row0 = sid * STRIPE

    for acc in accs:
        pltpu.sync_copy(y_hbm.at[cid, pl.ds(row0, STRIPE)],
                        acc.at[pl.ds(row0, STRIPE)])

        @pl.when(sid == 0)
        def _():
            pltpu.sync_copy(y_hbm.at[cid, pl.ds(NS * STRIPE, TAIL)],
                            acc.at[pl.ds(NS * STRIPE, TAIL)])

    plsc.subcore_barrier()

    for h in range(NHALF):
        acc = accs[h]
        pltpu.sync_copy(edge_hbm.at[0, sid, pl.ds(h * NCH_H, NCH_H)], src_v)
        pltpu.sync_copy(edge_hbm.at[1, sid, pl.ds(h * NCH_H, NCH_H)], dst_v)

        for b in range(NBUF):
            pltpu.async_copy(y_hbm.at[cid].at[src_v.at[b, 0]], rows[b], gsems[b])

        def body(g, _):
            for b in range(NBUF):
                j = g * NBUF + b
                pltpu.make_async_copy(y_hbm.at[cid].at[src_v.at[j, 0]],
                                      rows[b], gsems[b]).wait()
                pltpu.sync_copy(rows[b], acc.at[dst_v.at[j, 0]], add=True)

                @pl.when(j + NBUF < NCH_H)
                def _():
                    pltpu.async_copy(y_hbm.at[cid].at[src_v.at[j + NBUF, 0]],
                                     rows[b], gsems[b])
            return 0
        lax.fori_loop(0, NCH_H // NBUF, body, 0)

    plsc.subcore_barrier()
    for h in range(NHALF):
        pltpu.sync_copy(
            accs[h].at[pl.ds(row0, STRIPE)],
            out_hbm.at[cid, h, pl.ds(row0, STRIPE)],
        )

        @pl.when(sid == 0)
        def _():
            pltpu.sync_copy(
                accs[h].at[pl.ds(NS * STRIPE, TAIL)],
                out_hbm.at[cid, h, pl.ds(NS * STRIPE, TAIL)],
            )


_BM = 2000


def _dinv_from_parts(dp):
    deg = dp[0, :, 0:1] + dp[1, :, 0:1] + 1.0
    return lax.rsqrt(deg)


def _matmul_body(x_ref, w_ref, y_ref):
    y_ref[...] = jnp.dot(x_ref[...], w_ref[0],
                         preferred_element_type=jnp.float32)[None]


def _scale_body(xw_ref, dp_ref, y_ref):
    y_ref[...] = (xw_ref[...] * _dinv_from_parts(dp_ref[...])
                  ).astype(jnp.bfloat16)


def _finish_body(sp_ref, y_ref, dp_ref, b_ref, a_ref, o_ref):
    dinv = _dinv_from_parts(dp_ref[...])
    halves = [
        sp_ref[c, 0].astype(jnp.float32) + sp_ref[c, 1].astype(jnp.float32)
        - y_ref[c].astype(jnp.float32)
        for c in range(NC)
    ]
    o = jnp.concatenate(halves, axis=-1) * dinv + b_ref[...]
    o_ref[...] = jnp.where(o >= 0.0, o, a_ref[...] * o)


def kernel(x, edge_index, W, b, prelu_alpha):
    edges = edge_index.astype(jnp.int32).reshape(2, NS, NCH, 1, ECHUNK)

    deg_parts = _deg_kernel(edges)

    xw = pl.pallas_call(
        _matmul_body,
        grid=(NC, N // _BM),
        in_specs=[
            pl.BlockSpec((_BM, D), lambda h, i: (i, 0)),
            pl.BlockSpec((1, D, DH), lambda h, i: (h, 0, 0)),
        ],
        out_specs=pl.BlockSpec((1, _BM, DH), lambda h, i: (h, i, 0)),
        out_shape=jax.ShapeDtypeStruct((NC, N, DH), jnp.float32),
    )(x, W.reshape(D, NC, DH).transpose(1, 0, 2))

    y_halves = pl.pallas_call(
        _scale_body,
        grid=(NC, N // _BM),
        in_specs=[
            pl.BlockSpec((1, _BM, DH), lambda h, i: (h, i, 0)),
            pl.BlockSpec((NC, _BM, 16), lambda h, i: (0, i, 0)),
        ],
        out_specs=pl.BlockSpec((1, _BM, DH), lambda h, i: (h, i, 0)),
        out_shape=jax.ShapeDtypeStruct((NC, N, DH), jnp.bfloat16),
    )(xw, deg_parts)

    s_parts = _edge_kernel(y_halves, edges)

    out = pl.pallas_call(
        _finish_body,
        grid=(N // _BM,),
        in_specs=[
            pl.BlockSpec((NC, NHALF, _BM, DH), lambda i: (0, 0, i, 0)),
            pl.BlockSpec((NC, _BM, DH), lambda i: (0, i, 0)),
            pl.BlockSpec((NC, _BM, 16), lambda i: (0, i, 0)),
            pl.BlockSpec((D,), lambda i: (0,)),
            pl.BlockSpec((D,), lambda i: (0,)),
        ],
        out_specs=pl.BlockSpec((_BM, D), lambda i: (i, 0)),
        out_shape=jax.ShapeDtypeStruct((N, D), jnp.float32),
    )(s_parts, y_halves, deg_parts, b, prelu_alpha)

    return out


# speedup vs baseline: 1.1537x; 1.1537x over previous
import functools

import jax
import jax.numpy as jnp
from jax import lax
from jax.experimental import pallas as pl
from jax.experimental.pallas import tpu as pltpu
from jax.experimental.pallas import tpu_sc as plsc

N = 10000
D = 128
E = 320000
DH = D // 2

NC = 2
NS = 16
NW = NC * NS
ECHUNK = 1000
NCH_DEG = E // NW // ECHUNK
NCH = E // NS // ECHUNK
NHALF = 2
NCH_H = NCH // NHALF
NBUF = 2
KFIRE = 5
STRIPE = 624
TAIL = N - NS * STRIPE

_MESH = plsc.VectorSubcoreMesh(
    core_axis_name="c", subcore_axis_name="s", num_cores=NC, num_subcores=NS
)


@functools.partial(
    pl.kernel,
    out_type=jax.ShapeDtypeStruct((NC, N, 16), jnp.float32),
    mesh=_MESH,
    compiler_params=pltpu.CompilerParams(use_tc_tiling_on_sc=False),
    scratch_types=[
        pltpu.VMEM((STRIPE, 16), jnp.float32),
        pltpu.VMEM((ECHUNK, 16), jnp.float32),
        pltpu.VMEM((NCH_DEG, 1, ECHUNK), jnp.int32),
        pltpu.VMEM_SHARED((N, 16), jnp.float32),
        pltpu.SemaphoreType.DMA,
    ],
)
def _deg_kernel(edge_hbm, out_hbm, z_v, o_v, dst_v, acc, sem):
    cid = lax.axis_index("c")
    sid = lax.axis_index("s")
    row0 = sid * STRIPE

    def zbody(i, _):
        z_v[i] = jnp.zeros((16,), jnp.float32)
        return 0
    lax.fori_loop(0, STRIPE, zbody, 0)

    def obody(i, _):
        o_v[i] = jnp.ones((16,), jnp.float32)
        return 0
    lax.fori_loop(0, ECHUNK, obody, 0)

    pltpu.sync_copy(z_v, acc.at[pl.ds(row0, STRIPE)])

    @pl.when(sid == 0)
    def _():
        pltpu.sync_copy(z_v.at[pl.ds(0, TAIL)], acc.at[pl.ds(NS * STRIPE, TAIL)])

    pltpu.sync_copy(edge_hbm.at[1, sid, pl.ds(cid * NCH_DEG, NCH_DEG)], dst_v)
    plsc.subcore_barrier()

    for g in range(NCH_DEG // KFIRE):
        for k in range(KFIRE):
            pltpu.async_copy(o_v, acc.at[dst_v.at[g * KFIRE + k, 0]],
                             sem, add=True)
        for k in range(KFIRE):
            pltpu.make_async_copy(o_v, acc.at[dst_v.at[g * KFIRE + k, 0]],
                                  sem).wait()

    plsc.subcore_barrier()
    pltpu.sync_copy(
        acc.at[pl.ds(row0, STRIPE)],
        out_hbm.at[cid, pl.ds(row0, STRIPE)],
    )

    @pl.when(sid == 0)
    def _():
        pltpu.sync_copy(
            acc.at[pl.ds(NS * STRIPE, TAIL)],
            out_hbm.at[cid, pl.ds(NS * STRIPE, TAIL)],
        )


@functools.partial(
    pl.kernel,
    out_type=jax.ShapeDtypeStruct((NC, N, DH), jnp.bfloat16),
    mesh=_MESH,
    compiler_params=pltpu.CompilerParams(use_tc_tiling_on_sc=False),
    scratch_types=[
        pltpu.VMEM((NCH_H, 1, ECHUNK), jnp.int32),
        pltpu.VMEM((NCH_H, 1, ECHUNK), jnp.int32),
        [pltpu.VMEM((ECHUNK, DH), jnp.bfloat16) for _ in range(NBUF)],
        [pltpu.SemaphoreType.DMA for _ in range(NBUF)],
        pltpu.VMEM_SHARED((N, DH), jnp.bfloat16),
    ],
)
def _edge_kernel(y_hbm, edge_hbm, out_hbm,
                 src_v, dst_v, rows, gsems, acc):
    cid = lax.axis_index("c")
    sid = lax.axis_index("s")
    row0 = sid * STRIPE

    pltpu.sync_copy(y_hbm.at[cid, pl.ds(row0, STRIPE)],
                    acc.at[pl.ds(row0, STRIPE)])

    @pl.when(sid == 0)
    def _():
        pltpu.sync_copy(y_hbm.at[cid, pl.ds(NS * STRIPE, TAIL)],
                        acc.at[pl.ds(NS * STRIPE, TAIL)])

    plsc.subcore_barrier()

    for h in range(NHALF):
        pltpu.sync_copy(edge_hbm.at[0, sid, pl.ds(h * NCH_H, NCH_H)], src_v)
        pltpu.sync_copy(edge_hbm.at[1, sid, pl.ds(h * NCH_H, NCH_H)], dst_v)

        for b in range(NBUF):
            pltpu.async_copy(y_hbm.at[cid].at[src_v.at[b, 0]], rows[b], gsems[b])

        def body(g, _):
            for b in range(NBUF):
                j = g * NBUF + b
                pltpu.make_async_copy(y_hbm.at[cid].at[src_v.at[j, 0]],
                                      rows[b], gsems[b]).wait()
                pltpu.sync_copy(rows[b], acc.at[dst_v.at[j, 0]], add=True)

                @pl.when(j + NBUF < NCH_H)
                def _():
                    pltpu.async_copy(y_hbm.at[cid].at[src_v.at[j + NBUF, 0]],
                                     rows[b], gsems[b])
            return 0
        lax.fori_loop(0, NCH_H // NBUF, body, 0)

    plsc.subcore_barrier()
    pltpu.sync_copy(
        acc.at[pl.ds(row0, STRIPE)],
        out_hbm.at[cid, pl.ds(row0, STRIPE)],
    )

    @pl.when(sid == 0)
    def _():
        pltpu.sync_copy(
            acc.at[pl.ds(NS * STRIPE, TAIL)],
            out_hbm.at[cid, pl.ds(NS * STRIPE, TAIL)],
        )


_BM = 2000


def _dinv_from_parts(dp):
    deg = dp[0, :, 0:1] + dp[1, :, 0:1] + 1.0
    return lax.rsqrt(deg)


def _matmul_body(x_ref, w_ref, dp_ref, y_ref):
    xw = jnp.dot(x_ref[...], w_ref[0], preferred_element_type=jnp.float32)
    y_ref[...] = (xw * _dinv_from_parts(dp_ref[...]))[None].astype(jnp.bfloat16)


def _finish_body(sp_ref, dp_ref, b_ref, a_ref, o_ref):
    dinv = _dinv_from_parts(dp_ref[...])
    s = jnp.concatenate([sp_ref[0], sp_ref[1]], axis=-1).astype(jnp.float32)
    o = s * dinv + b_ref[...]
    o_ref[...] = jnp.where(o >= 0.0, o, a_ref[...] * o)


def kernel(x, edge_index, W, b, prelu_alpha):
    edges = edge_index.astype(jnp.int32).reshape(2, NS, NCH, 1, ECHUNK)

    deg_parts = _deg_kernel(edges)

    y_halves = pl.pallas_call(
        _matmul_body,
        grid=(NC, N // _BM),
        in_specs=[
            pl.BlockSpec((_BM, D), lambda h, i: (i, 0)),
            pl.BlockSpec((1, D, DH), lambda h, i: (h, 0, 0)),
            pl.BlockSpec((NC, _BM, 16), lambda h, i: (0, i, 0)),
        ],
        out_specs=pl.BlockSpec((1, _BM, DH), lambda h, i: (h, i, 0)),
        out_shape=jax.ShapeDtypeStruct((NC, N, DH), jnp.bfloat16),
    )(x, W.reshape(D, NC, DH).transpose(1, 0, 2), deg_parts)

    s_parts = _edge_kernel(y_halves, edges)

    out = pl.pallas_call(
        _finish_body,
        grid=(N // _BM,),
        in_specs=[
            pl.BlockSpec((NC, _BM, DH), lambda i: (0, i, 0)),
            pl.BlockSpec((NC, _BM, 16), lambda i: (0, i, 0)),
            pl.BlockSpec((D,), lambda i: (0,)),
            pl.BlockSpec((D,), lambda i: (0,)),
        ],
        out_specs=pl.BlockSpec((_BM, D), lambda i: (i, 0)),
        out_shape=jax.ShapeDtypeStruct((N, D), jnp.float32),
    )(s_parts, deg_parts, b, prelu_alpha)

    return out
